# baseline (speedup 1.0000x reference)
import functools

import jax
import jax.numpy as jnp
from jax.experimental import pallas as pl
from jax.experimental.pallas import tpu as pltpu

_NSLOT = 4
_NCHUNK = 1


def _fc_body(nu_ref, uniq_ref, start_ref, count_ref, order_ref,
             x_ref, w_hbm, b_ref, o_ref, wbuf, xt_ref, sem,
             *, B, S, T, D_OUT, D_IN):
    nu = nu_ref[0]
    rows = D_OUT // _NCHUNK

    def start_fetch(i):
        slot = jax.lax.rem(i, _NSLOT)
        e = uniq_ref[i]
        for c in range(_NCHUNK):
            pltpu.make_async_copy(
                w_hbm.at[e, pl.ds(c * rows, rows), :],
                wbuf.at[slot, pl.ds(c * rows, rows), :],
                sem.at[slot, c],
            ).start()

    def wait_fetch(i):
        slot = jax.lax.rem(i, _NSLOT)
        e = uniq_ref[i]
        for c in range(_NCHUNK):
            pltpu.make_async_copy(
                w_hbm.at[e, pl.ds(c * rows, rows), :],
                wbuf.at[slot, pl.ds(c * rows, rows), :],
                sem.at[slot, c],
            ).wait()

    start_fetch(0)

    @pl.when(nu > 1)
    def _pro2():
        start_fetch(1)

    def expert_body(i, carry):
        slot = jax.lax.rem(i, _NSLOT)

        @pl.when(i + 2 < nu)
        def _prefetch():
            start_fetch(i + 2)

        wait_fetch(i)

        e = uniq_ref[i]
        cnt = count_ref[i]
        st = start_ref[i]
        n_tiles = (cnt + T - 1) // T

        def tile_body(t, c2):
            base = st + t * T
            lim = st + cnt
            for k in range(T):
                pos = jnp.minimum(base + k, B - 1)
                src = order_ref[pos]
                xt_ref[k * S:(k + 1) * S, :] = x_ref[src]
            y = jax.lax.dot_general(
                xt_ref[:, :], wbuf[slot],
                dimension_numbers=(((1,), (1,)), ((), ())),
                preferred_element_type=jnp.float32,
            ) + b_ref[pl.ds(e, 1), :]
            for k in range(T):
                p = base + k

                @pl.when(p < lim)
                def _store(k=k, p=p):
                    dst = order_ref[jnp.minimum(p, B - 1)]
                    o_ref[dst] = y[k * S:(k + 1) * S, :]
            return c2

        jax.lax.fori_loop(0, n_tiles, tile_body, 0)
        return carry

    jax.lax.fori_loop(0, nu, expert_body, 0)


def kernel(x, indices, W, b):
    B, S, D_IN = x.shape
    E, D_OUT, _ = W.shape
    T = 8

    idx = indices.astype(jnp.int32)
    order = jnp.argsort(idx).astype(jnp.int32)
    sidx = jnp.take(idx, order)
    is_start = jnp.concatenate(
        [jnp.ones((1,), jnp.bool_), sidx[1:] != sidx[:-1]])
    slot = jnp.cumsum(is_start.astype(jnp.int32)) - 1
    nu = slot[-1:] + 1
    uniq_e = jnp.zeros((E,), jnp.int32).at[slot].set(sidx)
    count = jnp.zeros((E,), jnp.int32).at[slot].add(1)
    start = jnp.full((E,), B, jnp.int32).at[slot].min(
        jnp.arange(B, dtype=jnp.int32))

    return pl.pallas_call(
        functools.partial(_fc_body, B=B, S=S, T=T, D_OUT=D_OUT, D_IN=D_IN),
        in_specs=[
            pl.BlockSpec(memory_space=pltpu.SMEM),
            pl.BlockSpec(memory_space=pltpu.SMEM),
            pl.BlockSpec(memory_space=pltpu.SMEM),
            pl.BlockSpec(memory_space=pltpu.SMEM),
            pl.BlockSpec(memory_space=pltpu.SMEM),
            pl.BlockSpec(memory_space=pltpu.VMEM),
            pl.BlockSpec(memory_space=pl.ANY),
            pl.BlockSpec(memory_space=pltpu.VMEM),
        ],
        out_specs=pl.BlockSpec(memory_space=pltpu.VMEM),
        scratch_shapes=[
            pltpu.VMEM((_NSLOT, D_OUT, D_IN), jnp.float32),
            pltpu.VMEM((T * S, D_IN), jnp.float32),
            pltpu.SemaphoreType.DMA((_NSLOT, _NCHUNK)),
        ],
        out_shape=jax.ShapeDtypeStruct((B, S, D_OUT), jnp.float32),
    )(nu, uniq_e, start, count, order, x, W, b)


# speedup vs baseline: 1.5789x; 1.5789x over previous
import functools

import jax
import jax.numpy as jnp
from jax.experimental import pallas as pl
from jax.experimental.pallas import tpu as pltpu

_NSLOT = 3


def _fc_body(idx_ref, x_ref, w_hbm, b_ref, o_ref,
             wbuf, xt_ref, uniq, ustart, ucnt, order, cursor, sem,
             *, B, S, T, E):
    def zero_body(e, c2):
        cursor[e] = 0
        return c2

    jax.lax.fori_loop(0, E, zero_body, 0)

    def cnt_body(i, c2):
        e = idx_ref[i]
        cursor[e] = cursor[e] + 1
        return c2

    jax.lax.fori_loop(0, B, cnt_body, 0)

    def pre_body(e, carry):
        pos, k = carry
        c = cursor[e]

        @pl.when(c > 0)
        def _emit():
            uniq[k] = e
            ustart[k] = pos
            ucnt[k] = c

        cursor[e] = pos
        return (pos + c, jnp.where(c > 0, k + 1, k))

    _, nu = jax.lax.fori_loop(0, E, pre_body, (0, 0))

    def ord_body(i, c2):
        e = idx_ref[i]
        p = cursor[e]
        order[p] = i
        cursor[e] = p + 1
        return c2

    jax.lax.fori_loop(0, B, ord_body, 0)

    def start_fetch(i):
        slot = jax.lax.rem(i, _NSLOT)
        pltpu.make_async_copy(
            w_hbm.at[uniq[i]], wbuf.at[slot], sem.at[slot]).start()

    def wait_fetch(i):
        slot = jax.lax.rem(i, _NSLOT)
        pltpu.make_async_copy(
            w_hbm.at[uniq[i]], wbuf.at[slot], sem.at[slot]).wait()

    start_fetch(0)

    @pl.when(nu > 1)
    def _pro2():
        start_fetch(1)

    def expert_body(i, carry):
        slot = jax.lax.rem(i, _NSLOT)

        @pl.when(i + 2 < nu)
        def _prefetch():
            start_fetch(i + 2)

        wait_fetch(i)

        e = uniq[i]
        cnt = ucnt[i]
        st = ustart[i]
        n_tiles = (cnt + T - 1) // T

        def tile_body(t, c2):
            base = st + t * T
            lim = st + cnt
            for k in range(T):
                pos = jnp.minimum(base + k, B - 1)
                src = order[pos]
                xt_ref[k * S:(k + 1) * S, :] = x_ref[src]
            y = jax.lax.dot_general(
                xt_ref[:, :], wbuf[slot],
                dimension_numbers=(((1,), (1,)), ((), ())),
                preferred_element_type=jnp.float32,
            ) + b_ref[pl.ds(e, 1), :]
            for k in range(T):
                p = base + k

                @pl.when(p < lim)
                def _store(k=k, p=p):
                    dst = order[jnp.minimum(p, B - 1)]
                    o_ref[dst] = y[k * S:(k + 1) * S, :]
            return c2

        jax.lax.fori_loop(0, n_tiles, tile_body, 0)
        return carry

    jax.lax.fori_loop(0, nu, expert_body, 0)


def kernel(x, indices, W, b):
    B, S, D_IN = x.shape
    E, D_OUT, _ = W.shape
    T = 8

    return pl.pallas_call(
        functools.partial(_fc_body, B=B, S=S, T=T, E=E),
        in_specs=[
            pl.BlockSpec(memory_space=pltpu.SMEM),
            pl.BlockSpec(memory_space=pltpu.VMEM),
            pl.BlockSpec(memory_space=pl.ANY),
            pl.BlockSpec(memory_space=pltpu.VMEM),
        ],
        out_specs=pl.BlockSpec(memory_space=pltpu.VMEM),
        scratch_shapes=[
            pltpu.VMEM((_NSLOT, D_OUT, D_IN), jnp.float32),
            pltpu.VMEM((T * S, D_IN), jnp.float32),
            pltpu.SMEM((E,), jnp.int32),
            pltpu.SMEM((E,), jnp.int32),
            pltpu.SMEM((E,), jnp.int32),
            pltpu.SMEM((B,), jnp.int32),
            pltpu.SMEM((E,), jnp.int32),
            pltpu.SemaphoreType.DMA((_NSLOT,)),
        ],
        out_shape=jax.ShapeDtypeStruct((B, S, D_OUT), jnp.float32),
    )(indices.astype(jnp.int32), x, W, b)


# speedup vs baseline: 1.5992x; 1.0129x over previous
import functools

import jax
import jax.numpy as jnp
from jax.experimental import pallas as pl
from jax.experimental.pallas import tpu as pltpu

_NSLOT = 3


def _fc_body(idx_ref, x_hbm, w_hbm, b_ref, o_ref,
             wbuf, xv_ref, xt_ref, slotof, uniq, ustart, ucnt, order, cursor,
             sem, xsem, *, B, S, T, E):
    def start_fetch_expert(e, slot):
        pltpu.make_async_copy(
            w_hbm.at[e], wbuf.at[slot], sem.at[slot]).start()

    def start_fetch(i):
        start_fetch_expert(uniq[i], jax.lax.rem(i, _NSLOT))

    def wait_fetch(i):
        slot = jax.lax.rem(i, _NSLOT)
        pltpu.make_async_copy(
            w_hbm.at[uniq[i]], wbuf.at[slot], sem.at[slot]).wait()

    start_fetch_expert(idx_ref[0], 0)
    xcopy = pltpu.make_async_copy(x_hbm, xv_ref, xsem)
    xcopy.start()

    def clear_body(e, c2):
        slotof[e] = -1
        return c2

    jax.lax.fori_loop(0, E, clear_body, 0)

    def cnt_body(i, k):
        e = idx_ref[i]
        s = slotof[e]
        fresh = s < 0

        @pl.when(fresh)
        def _new():
            slotof[e] = k
            uniq[k] = e
            ucnt[k] = 1

        @pl.when(jnp.logical_not(fresh))
        def _old():
            ucnt[s] = ucnt[s] + 1

        return jnp.where(fresh, k + 1, k)

    nu = jax.lax.fori_loop(0, B, cnt_body, 0)

    def pre_body(k, pos):
        ustart[k] = pos
        cursor[k] = pos
        return pos + ucnt[k]

    jax.lax.fori_loop(0, nu, pre_body, 0)

    def ord_body(i, c2):
        k = slotof[idx_ref[i]]
        p = cursor[k]
        order[p] = i
        cursor[k] = p + 1
        return c2

    jax.lax.fori_loop(0, B, ord_body, 0)

    @pl.when(nu > 1)
    def _pro2():
        start_fetch(1)

    xcopy.wait()

    def expert_body(i, carry):
        slot = jax.lax.rem(i, _NSLOT)

        @pl.when(i + 2 < nu)
        def _prefetch():
            start_fetch(i + 2)

        wait_fetch(i)

        e = uniq[i]
        cnt = ucnt[i]
        st = ustart[i]
        n_tiles = (cnt + T - 1) // T

        def tile_body(t, c2):
            base = st + t * T
            lim = st + cnt
            for k in range(T):
                pos = jnp.minimum(base + k, B - 1)
                src = order[pos]
                xt_ref[k * S:(k + 1) * S, :] = xv_ref[src]
            y = jax.lax.dot_general(
                xt_ref[:, :], wbuf[slot],
                dimension_numbers=(((1,), (1,)), ((), ())),
                preferred_element_type=jnp.float32,
            ) + b_ref[pl.ds(e, 1), :]
            for k in range(T):
                p = base + k

                @pl.when(p < lim)
                def _store(k=k, p=p):
                    dst = order[jnp.minimum(p, B - 1)]
                    o_ref[dst] = y[k * S:(k + 1) * S, :]
            return c2

        jax.lax.fori_loop(0, n_tiles, tile_body, 0)
        return carry

    jax.lax.fori_loop(0, nu, expert_body, 0)


def kernel(x, indices, W, b):
    B, S, D_IN = x.shape
    E, D_OUT, _ = W.shape
    T = 8

    return pl.pallas_call(
        functools.partial(_fc_body, B=B, S=S, T=T, E=E),
        in_specs=[
            pl.BlockSpec(memory_space=pltpu.SMEM),
            pl.BlockSpec(memory_space=pl.ANY),
            pl.BlockSpec(memory_space=pl.ANY),
            pl.BlockSpec(memory_space=pltpu.VMEM),
        ],
        out_specs=pl.BlockSpec(memory_space=pltpu.VMEM),
        scratch_shapes=[
            pltpu.VMEM((_NSLOT, D_OUT, D_IN), jnp.float32),
            pltpu.VMEM((B, S, D_IN), jnp.float32),
            pltpu.VMEM((T * S, D_IN), jnp.float32),
            pltpu.SMEM((E,), jnp.int32),
            pltpu.SMEM((E,), jnp.int32),
            pltpu.SMEM((E,), jnp.int32),
            pltpu.SMEM((E,), jnp.int32),
            pltpu.SMEM((B,), jnp.int32),
            pltpu.SMEM((E,), jnp.int32),
            pltpu.SemaphoreType.DMA((_NSLOT,)),
            pltpu.SemaphoreType.DMA,
        ],
        out_shape=jax.ShapeDtypeStruct((B, S, D_OUT), jnp.float32),
    )(indices.astype(jnp.int32), x, W, b)
